# Initial kernel scaffold; baseline (speedup 1.0000x reference)
import jax
import jax.numpy as jnp
from jax.experimental import pallas as pl


def kernel(x, edge_index, edge_weight, W, b):
    raise NotImplementedError("write your pallas kernel here")


# speedup vs baseline: 2.3998x; 2.3998x over previous
import functools

import jax
import jax.numpy as jnp
from jax import lax
from jax.experimental import pallas as pl
from jax.experimental.pallas import tpu as pltpu
from jax.experimental.pallas import tpu_sc as plsc

N = 10000
D = 128
E = 320000
NC = 2
NS = 16
L = 16
NW = NC * NS
CHUNK = 128
CHUNKS_PER_W = 80
EDGES_PER_W = CHUNK * CHUNKS_PER_W
E_PAD = NW * EDGES_PER_W
N_PAD = 10240
ROWS_PER_TILE = N_PAD // NS

_mesh = plsc.VectorSubcoreMesh(core_axis_name="c", subcore_axis_name="s")


@functools.partial(
    pl.kernel,
    mesh=_mesh,
    out_type=jax.ShapeDtypeStruct((NC, N_PAD, D), jnp.float32),
    scratch_types=[
        pltpu.VMEM((CHUNK,), jnp.int32),
        pltpu.VMEM((CHUNK,), jnp.int32),
        pltpu.VMEM((CHUNK, L), jnp.float32),
        pltpu.VMEM((CHUNK, D), jnp.float32),
        pltpu.VMEM_SHARED((N_PAD, D), jnp.float32),
        pltpu.SemaphoreType.DMA,
    ],
)
def _aggregate(x_hbm, src_hbm, dst_hbm, w_hbm, out_hbm,
               src_v, dst_v, w_v, rows_v, acc, sem):
    cid = lax.axis_index("c")
    sid = lax.axis_index("s")
    wid = cid * NS + sid

    zeros16 = jnp.zeros((L,), jnp.float32)

    def _zero_rows(r, _):
        for c in range(D // L):
            rows_v[r, pl.ds(c * L, L)] = zeros16
        return 0

    lax.fori_loop(0, CHUNK, _zero_rows, 0)

    rbase = sid * ROWS_PER_TILE
    for k in range(ROWS_PER_TILE // CHUNK):
        pltpu.sync_copy(rows_v, acc.at[pl.ds(rbase + k * CHUNK, CHUNK)])
    plsc.subcore_barrier()

    ebase = wid * EDGES_PER_W

    def _chunk(ci, _):
        base = ebase + ci * CHUNK
        pltpu.sync_copy(src_hbm.at[pl.ds(base, CHUNK)], src_v)
        pltpu.sync_copy(dst_hbm.at[pl.ds(base, CHUNK)], dst_v)
        pltpu.sync_copy(w_hbm.at[pl.ds(base, CHUNK)], w_v)
        pltpu.async_copy(x_hbm.at[src_v], rows_v, sem).wait()

        def _scale(e, _):
            wb = w_v[e, :]
            for c in range(D // L):
                rows_v[e, pl.ds(c * L, L)] = rows_v[e, pl.ds(c * L, L)] * wb
            return 0

        lax.fori_loop(0, CHUNK, _scale, 0)
        pltpu.sync_copy(rows_v, acc.at[dst_v], add=True)
        return 0

    lax.fori_loop(0, CHUNKS_PER_W, _chunk, 0)
    plsc.subcore_barrier()

    pltpu.sync_copy(acc.at[pl.ds(rbase, ROWS_PER_TILE)],
                    out_hbm.at[cid, pl.ds(rbase, ROWS_PER_TILE)])


_BLK = 1000


def _mm_body(p_ref, w_ref, b_ref, o_ref):
    s = p_ref[0] + p_ref[1]
    o_ref[...] = (
        jnp.dot(s, w_ref[...], preferred_element_type=jnp.float32) + b_ref[...]
    )


_matmul = pl.pallas_call(
    _mm_body,
    grid=(N // _BLK,),
    in_specs=[
        pl.BlockSpec((NC, _BLK, D), lambda i: (0, i, 0)),
        pl.BlockSpec((D, D), lambda i: (0, 0)),
        pl.BlockSpec((1, D), lambda i: (0, 0)),
    ],
    out_specs=pl.BlockSpec((_BLK, D), lambda i: (i, 0)),
    out_shape=jax.ShapeDtypeStruct((N, D), jnp.float32),
)


def kernel(x, edge_index, edge_weight, W, b):
    pad = E_PAD - E
    src = jnp.concatenate([edge_index[0], jnp.zeros((pad,), jnp.int32)])
    dst = jnp.concatenate([edge_index[1], jnp.zeros((pad,), jnp.int32)])
    w = jnp.concatenate([edge_weight, jnp.zeros((pad,), jnp.float32)])
    w = jnp.broadcast_to(w[:, None], (E_PAD, L))
    partials = _aggregate(x, src, dst, w)
    return _matmul(partials, W, b.reshape(1, D))


# speedup vs baseline: 2.9758x; 1.2400x over previous
import functools

import jax
import jax.numpy as jnp
from jax import lax
from jax.experimental import pallas as pl
from jax.experimental.pallas import tpu as pltpu
from jax.experimental.pallas import tpu_sc as plsc

N = 10000
D = 128
E = 320000
NC = 2
NS = 16
L = 16
NW = NC * NS
CHUNK = 64
CHUNKS_PER_W = 160
EDGES_PER_W = CHUNK * CHUNKS_PER_W
E_PAD = NW * EDGES_PER_W
N_PAD = 10240
ROWS_PER_TILE = N_PAD // NS

_mesh = plsc.VectorSubcoreMesh(core_axis_name="c", subcore_axis_name="s")


@functools.partial(
    pl.kernel,
    mesh=_mesh,
    out_type=jax.ShapeDtypeStruct((NC, N_PAD, D), jnp.float32),
    scratch_types=[
        pltpu.VMEM((CHUNK,), jnp.int32),
        pltpu.VMEM((CHUNK,), jnp.int32),
        pltpu.VMEM((CHUNK, L), jnp.float32),
        pltpu.VMEM((CHUNK, D), jnp.float32),
        pltpu.VMEM((CHUNK,), jnp.int32),
        pltpu.VMEM((CHUNK,), jnp.int32),
        pltpu.VMEM((CHUNK, L), jnp.float32),
        pltpu.VMEM((CHUNK, D), jnp.float32),
        pltpu.VMEM_SHARED((N_PAD, D), jnp.float32),
        pltpu.SemaphoreType.DMA,
        pltpu.SemaphoreType.DMA,
    ],
)
def _aggregate(x_hbm, src_hbm, dst_hbm, w_hbm, out_hbm,
               src0, dst0, w0, rows0, src1, dst1, w1, rows1,
               acc, gsem0, gsem1):
    cid = lax.axis_index("c")
    sid = lax.axis_index("s")
    wid = cid * NS + sid
    bufs = ((src0, dst0, w0, rows0, gsem0), (src1, dst1, w1, rows1, gsem1))

    zeros16 = jnp.zeros((L,), jnp.float32)

    def _zero_rows(r, _):
        for c in range(D // L):
            rows0[r, pl.ds(c * L, L)] = zeros16
        return 0

    lax.fori_loop(0, CHUNK, _zero_rows, 0)

    rbase = sid * ROWS_PER_TILE
    for k in range(ROWS_PER_TILE // CHUNK):
        pltpu.sync_copy(rows0, acc.at[pl.ds(rbase + k * CHUNK, CHUNK)])
    plsc.subcore_barrier()

    ebase = wid * EDGES_PER_W

    def _load_and_fire(c, b):
        src_v, dst_v, w_v, rows_v, gsem = bufs[b]
        base = ebase + c * CHUNK
        pltpu.sync_copy(src_hbm.at[pl.ds(base, CHUNK)], src_v)
        pltpu.sync_copy(dst_hbm.at[pl.ds(base, CHUNK)], dst_v)
        pltpu.sync_copy(w_hbm.at[pl.ds(base, CHUNK)], w_v)
        pltpu.async_copy(x_hbm.at[src_v], rows_v, gsem)

    def _process(c, b, prefetch):
        src_v, dst_v, w_v, rows_v, gsem = bufs[b]
        pltpu.make_async_copy(x_hbm.at[src_v], rows_v, gsem).wait()

        def _scale(e, _):
            wb = w_v[e, :]
            for cc in range(D // L):
                rows_v[e, pl.ds(cc * L, L)] = rows_v[e, pl.ds(cc * L, L)] * wb
            return 0

        lax.fori_loop(0, CHUNK, _scale, 0)
        pltpu.sync_copy(rows_v, acc.at[dst_v], add=True)
        if prefetch:
            _load_and_fire(c + 2, b)

    _load_and_fire(0, 0)
    _load_and_fire(1, 1)

    def _outer(o, _):
        c = o * 2
        _process(c, 0, True)
        _process(c + 1, 1, True)
        return 0

    lax.fori_loop(0, CHUNKS_PER_W // 2 - 1, _outer, 0)
    _process(CHUNKS_PER_W - 2, 0, False)
    _process(CHUNKS_PER_W - 1, 1, False)
    plsc.subcore_barrier()

    pltpu.sync_copy(acc.at[pl.ds(rbase, ROWS_PER_TILE)],
                    out_hbm.at[cid, pl.ds(rbase, ROWS_PER_TILE)])


_BLK = 1000


def _mm_body(p_ref, w_ref, b_ref, o_ref):
    s = p_ref[0] + p_ref[1]
    o_ref[...] = (
        jnp.dot(s, w_ref[...], preferred_element_type=jnp.float32) + b_ref[...]
    )


_matmul = pl.pallas_call(
    _mm_body,
    grid=(N // _BLK,),
    in_specs=[
        pl.BlockSpec((NC, _BLK, D), lambda i: (0, i, 0)),
        pl.BlockSpec((D, D), lambda i: (0, 0)),
        pl.BlockSpec((1, D), lambda i: (0, 0)),
    ],
    out_specs=pl.BlockSpec((_BLK, D), lambda i: (i, 0)),
    out_shape=jax.ShapeDtypeStruct((N, D), jnp.float32),
)


def kernel(x, edge_index, edge_weight, W, b):
    pad = E_PAD - E
    src = jnp.concatenate([edge_index[0], jnp.zeros((pad,), jnp.int32)])
    dst = jnp.concatenate([edge_index[1], jnp.zeros((pad,), jnp.int32)])
    w = jnp.concatenate([edge_weight, jnp.zeros((pad,), jnp.float32)])
    w = jnp.broadcast_to(w[:, None], (E_PAD, L))
    partials = _aggregate(x, src, dst, w)
    return _matmul(partials, W, b.reshape(1, D))


# speedup vs baseline: 6.2852x; 2.1121x over previous
import functools

import jax
import jax.numpy as jnp
from jax import lax
from jax.experimental import pallas as pl
from jax.experimental.pallas import tpu as pltpu
from jax.experimental.pallas import tpu_sc as plsc

N = 10000
D = 128
E = 320000
NC = 2
NS = 16
L = 16
NW = NC * NS
CHUNK = 80
CHUNKS_PER_W = 125
EDGES_PER_W = CHUNK * CHUNKS_PER_W
N_PAD = 10240
ROWS_PER_TILE = N_PAD // NS

_mesh = plsc.VectorSubcoreMesh(core_axis_name="c", subcore_axis_name="s")


@functools.partial(
    pl.kernel,
    mesh=_mesh,
    out_type=jax.ShapeDtypeStruct((NC, N_PAD, D), jnp.float32),
    scratch_types=[
        pltpu.VMEM((CHUNK,), jnp.int32),
        pltpu.VMEM((CHUNK,), jnp.int32),
        pltpu.VMEM((CHUNK,), jnp.float32),
        pltpu.VMEM((CHUNK, D), jnp.float32),
        pltpu.VMEM((CHUNK,), jnp.int32),
        pltpu.VMEM((CHUNK,), jnp.int32),
        pltpu.VMEM((CHUNK,), jnp.float32),
        pltpu.VMEM((CHUNK, D), jnp.float32),
        pltpu.VMEM_SHARED((N_PAD, D), jnp.float32),
        pltpu.SemaphoreType.DMA,
        pltpu.SemaphoreType.DMA,
    ],
)
def _aggregate(x_hbm, src_hbm, dst_hbm, w_hbm, out_hbm,
               src0, dst0, w0, rows0, src1, dst1, w1, rows1,
               acc, gsem0, gsem1):
    cid = lax.axis_index("c")
    sid = lax.axis_index("s")
    wid = cid * NS + sid
    bufs = ((src0, dst0, w0, rows0, gsem0), (src1, dst1, w1, rows1, gsem1))

    zeros16 = jnp.zeros((L,), jnp.float32)

    def _zero_rows(r, _):
        for c in range(D // L):
            rows0[r, pl.ds(c * L, L)] = zeros16
        return 0

    lax.fori_loop(0, CHUNK, _zero_rows, 0)

    rbase = sid * ROWS_PER_TILE
    for k in range(ROWS_PER_TILE // CHUNK):
        pltpu.sync_copy(rows0, acc.at[pl.ds(rbase + k * CHUNK, CHUNK)])
    plsc.subcore_barrier()

    ebase = wid * EDGES_PER_W

    def _load_and_fire(c, b):
        src_v, dst_v, w_v, rows_v, gsem = bufs[b]
        base = ebase + c * CHUNK
        pltpu.sync_copy(src_hbm.at[pl.ds(base, CHUNK)], src_v)
        pltpu.sync_copy(dst_hbm.at[pl.ds(base, CHUNK)], dst_v)
        pltpu.sync_copy(w_hbm.at[pl.ds(base, CHUNK)], w_v)
        pltpu.async_copy(x_hbm.at[src_v], rows_v, gsem)

    def _process(c, b, prefetch):
        src_v, dst_v, w_v, rows_v, gsem = bufs[b]
        pltpu.make_async_copy(x_hbm.at[src_v], rows_v, gsem).wait()

        for g in range(CHUNK // L):
            w16 = w_v[pl.ds(g * L, L)]
            for e in range(L):
                wb = jnp.broadcast_to(w16[e], (L,))
                r = g * L + e
                for cc in range(D // L):
                    rows_v[r, pl.ds(cc * L, L)] = (
                        rows_v[r, pl.ds(cc * L, L)] * wb
                    )

        pltpu.sync_copy(rows_v, acc.at[dst_v], add=True)
        if prefetch:
            _load_and_fire(c + 2, b)

    _load_and_fire(0, 0)
    _load_and_fire(1, 1)

    def _outer(o, _):
        c = o * 2
        _process(c, 0, True)
        _process(c + 1, 1, True)
        return 0

    lax.fori_loop(0, CHUNKS_PER_W // 2 - 1, _outer, 0)
    _process(CHUNKS_PER_W - 3, 0, True)
    _process(CHUNKS_PER_W - 2, 1, False)
    _process(CHUNKS_PER_W - 1, 0, False)
    plsc.subcore_barrier()

    pltpu.sync_copy(acc.at[pl.ds(rbase, ROWS_PER_TILE)],
                    out_hbm.at[cid, pl.ds(rbase, ROWS_PER_TILE)])


_BLK = 1000


def _mm_body(p_ref, w_ref, b_ref, o_ref):
    s = p_ref[0] + p_ref[1]
    o_ref[...] = (
        jnp.dot(s, w_ref[...], preferred_element_type=jnp.float32) + b_ref[...]
    )


_matmul = pl.pallas_call(
    _mm_body,
    grid=(N // _BLK,),
    in_specs=[
        pl.BlockSpec((NC, _BLK, D), lambda i: (0, i, 0)),
        pl.BlockSpec((D, D), lambda i: (0, 0)),
        pl.BlockSpec((1, D), lambda i: (0, 0)),
    ],
    out_specs=pl.BlockSpec((_BLK, D), lambda i: (i, 0)),
    out_shape=jax.ShapeDtypeStruct((N, D), jnp.float32),
)


def kernel(x, edge_index, edge_weight, W, b):
    partials = _aggregate(x, edge_index[0], edge_index[1], edge_weight)
    return _matmul(partials, W, b.reshape(1, D))


# speedup vs baseline: 7.9201x; 1.2601x over previous
import functools

import jax
import jax.numpy as jnp
from jax import lax
from jax.experimental import pallas as pl
from jax.experimental.pallas import tpu as pltpu
from jax.experimental.pallas import tpu_sc as plsc

N = 10000
D = 128
E = 320000
NC = 2
NS = 16
L = 16
NW = NC * NS
CHUNK = 80
CHUNKS_PER_W = 125
EDGES_PER_W = CHUNK * CHUNKS_PER_W
N_PAD = 10240
ROWS_PER_TILE = N_PAD // NS
NIB = 4
LAST = CHUNKS_PER_W - 1

_mesh = plsc.VectorSubcoreMesh(core_axis_name="c", subcore_axis_name="s")


@functools.partial(
    pl.kernel,
    mesh=_mesh,
    out_type=jax.ShapeDtypeStruct((NC, N_PAD, D), jnp.float32),
    scratch_types=[
        pltpu.VMEM((2, CHUNK), jnp.int32),
        pltpu.VMEM((2, CHUNK), jnp.int32),
        pltpu.VMEM((2, CHUNK), jnp.int32),
        pltpu.VMEM((2, CHUNK), jnp.int32),
        pltpu.VMEM((CHUNK,), jnp.float32),
        pltpu.VMEM((CHUNK,), jnp.float32),
        pltpu.VMEM((CHUNK,), jnp.float32),
        pltpu.VMEM((CHUNK,), jnp.float32),
        pltpu.VMEM((CHUNK, D), jnp.float32),
        pltpu.VMEM((CHUNK, D), jnp.float32),
        pltpu.VMEM_SHARED((N_PAD, D), jnp.float32),
        pltpu.SemaphoreType.DMA,
        pltpu.SemaphoreType.DMA,
        pltpu.SemaphoreType.DMA,
        pltpu.SemaphoreType.DMA,
        pltpu.SemaphoreType.DMA,
        pltpu.SemaphoreType.DMA,
    ],
)
def _aggregate(x_hbm, idx_hbm, w_hbm, out_hbm,
               idx0, idx1, idx2, idx3, w0, w1, w2, w3, rows0, rows1,
               acc, gsem0, gsem1, isem0, isem1, isem2, isem3):
    cid = lax.axis_index("c")
    sid = lax.axis_index("s")
    wid = cid * NS + sid
    idxb = (idx0, idx1, idx2, idx3)
    wvb = (w0, w1, w2, w3)
    isems = (isem0, isem1, isem2, isem3)
    rowsb = (rows0, rows1)
    gsems = (gsem0, gsem1)

    zeros16 = jnp.zeros((L,), jnp.float32)

    def _zero_rows(r, _):
        for c in range(D // L):
            rows0[r, pl.ds(c * L, L)] = zeros16
        return 0

    lax.fori_loop(0, CHUNK, _zero_rows, 0)

    rbase = sid * ROWS_PER_TILE
    for k in range(ROWS_PER_TILE // CHUNK):
        pltpu.sync_copy(rows0, acc.at[pl.ds(rbase + k * CHUNK, CHUNK)])
    plsc.subcore_barrier()

    ebase = wid * EDGES_PER_W

    cbase = wid * CHUNKS_PER_W

    def _fire_idx(c, i):
        pltpu.async_copy(idx_hbm.at[cbase + c], idxb[i], isems[i])
        pltpu.async_copy(w_hbm.at[cbase + c], wvb[i], isems[i])

    def _wait_idx(c, i):
        pltpu.make_async_copy(idx_hbm.at[cbase + c], idxb[i], isems[i]).wait()
        pltpu.make_async_copy(w_hbm.at[cbase + c], wvb[i], isems[i]).wait()

    def _fire_gather(i, rb):
        pltpu.async_copy(x_hbm.at[idxb[i].at[0]], rowsb[rb], gsems[rb])

    def _wait_gather(i, rb):
        pltpu.make_async_copy(x_hbm.at[idxb[i].at[0]], rowsb[rb],
                              gsems[rb]).wait()

    def _scale_scatter(i, rb):
        rows_v = rowsb[rb]
        w_row = idxb[i]
        w_v = wvb[i]
        for g in range(CHUNK // L):
            w16 = w_v[pl.ds(g * L, L)]
            for e in range(L):
                wb = jnp.broadcast_to(w16[e], (L,))
                r = g * L + e
                for cc in range(D // L):
                    rows_v[r, pl.ds(cc * L, L)] = (
                        rows_v[r, pl.ds(cc * L, L)] * wb
                    )
        pltpu.sync_copy(rows_v, acc.at[w_row.at[1]], add=True)

    for i in range(NIB):
        _fire_idx(i, i)
    _wait_idx(0, 0)
    _fire_gather(0, 0)

    def _outer(o, _):
        c0 = o * 4
        for j in range(4):
            c = c0 + j
            i = j
            rb = j & 1
            _wait_idx(c + 1, (j + 1) % 4)
            _fire_gather((j + 1) % 4, (j + 1) & 1)
            _wait_gather(i, rb)
            _scale_scatter(i, rb)
            _fire_idx(jnp.minimum(c + 4, LAST), i)
        return 0

    lax.fori_loop(0, (CHUNKS_PER_W - 1) // 4, _outer, 0)

    _wait_gather(0, 0)
    _scale_scatter(0, 0)
    for i in range(1, NIB):
        _wait_idx(LAST, i)
    plsc.subcore_barrier()

    pltpu.sync_copy(acc.at[pl.ds(rbase, ROWS_PER_TILE)],
                    out_hbm.at[cid, pl.ds(rbase, ROWS_PER_TILE)])


_BLK = 1000


def _mm_body(p_ref, w_ref, b_ref, o_ref):
    s = p_ref[0] + p_ref[1]
    o_ref[...] = (
        jnp.dot(s, w_ref[...], preferred_element_type=jnp.float32) + b_ref[...]
    )


_matmul = pl.pallas_call(
    _mm_body,
    grid=(N // _BLK,),
    in_specs=[
        pl.BlockSpec((NC, _BLK, D), lambda i: (0, i, 0)),
        pl.BlockSpec((D, D), lambda i: (0, 0)),
        pl.BlockSpec((1, D), lambda i: (0, 0)),
    ],
    out_specs=pl.BlockSpec((_BLK, D), lambda i: (i, 0)),
    out_shape=jax.ShapeDtypeStruct((N, D), jnp.float32),
)


def kernel(x, edge_index, edge_weight, W, b):
    nwc = NW * CHUNKS_PER_W
    idx = edge_index.reshape(2, nwc, CHUNK).transpose(1, 0, 2)
    wc = edge_weight.reshape(nwc, CHUNK)
    partials = _aggregate(x, idx, wc)
    return _matmul(partials, W, b.reshape(1, D))


# speedup vs baseline: 9.4985x; 1.1993x over previous
import functools

import jax
import jax.numpy as jnp
from jax import lax
from jax.experimental import pallas as pl
from jax.experimental.pallas import tpu as pltpu
from jax.experimental.pallas import tpu_sc as plsc

N = 10000
D = 128
E = 320000
NC = 2
NS = 16
L = 16
NW = NC * NS
CHUNK = 80
CHUNKS_PER_W = 125
EDGES_PER_W = CHUNK * CHUNKS_PER_W
N_PAD = 10240
ROWS_PER_TILE = N_PAD // NS
NR = 3
LAST = CHUNKS_PER_W - 1

_mesh = plsc.VectorSubcoreMesh(core_axis_name="c", subcore_axis_name="s")


@functools.partial(
    pl.kernel,
    mesh=_mesh,
    out_type=jax.ShapeDtypeStruct((NC, N_PAD, D), jnp.float32),
    scratch_types=[
        pltpu.VMEM((2, CHUNK), jnp.int32),
        pltpu.VMEM((2, CHUNK), jnp.int32),
        pltpu.VMEM((2, CHUNK), jnp.int32),
        pltpu.VMEM((CHUNK,), jnp.float32),
        pltpu.VMEM((CHUNK,), jnp.float32),
        pltpu.VMEM((CHUNK,), jnp.float32),
        pltpu.VMEM((1, CHUNK), jnp.int32),
        pltpu.VMEM((1, CHUNK), jnp.int32),
        pltpu.VMEM((1, CHUNK), jnp.int32),
        pltpu.VMEM((CHUNK, D), jnp.float32),
        pltpu.VMEM((CHUNK, D), jnp.float32),
        pltpu.VMEM((CHUNK, D), jnp.float32),
        pltpu.VMEM_SHARED((N_PAD, D), jnp.float32),
        pltpu.SemaphoreType.DMA,
        pltpu.SemaphoreType.DMA,
        pltpu.SemaphoreType.DMA,
        pltpu.SemaphoreType.DMA,
        pltpu.SemaphoreType.DMA,
        pltpu.SemaphoreType.DMA,
        pltpu.SemaphoreType.DMA,
        pltpu.SemaphoreType.DMA,
        pltpu.SemaphoreType.DMA,
    ],
)
def _aggregate(x_hbm, idx_hbm, w_hbm, out_hbm,
               idx0, idx1, idx2, w0, w1, w2, dst0, dst1, dst2,
               rows0, rows1, rows2, acc,
               gsem0, gsem1, gsem2, isem0, isem1, isem2,
               ssem0, ssem1, ssem2):
    cid = lax.axis_index("c")
    sid = lax.axis_index("s")
    wid = cid * NS + sid
    idxb = (idx0, idx1, idx2)
    wvb = (w0, w1, w2)
    dstb = (dst0, dst1, dst2)
    rowsb = (rows0, rows1, rows2)
    isems = (isem0, isem1, isem2)
    gsems = (gsem0, gsem1, gsem2)
    ssems = (ssem0, ssem1, ssem2)

    zeros16 = jnp.zeros((L,), jnp.float32)

    def _zero_rows(r, _):
        for c in range(D // L):
            rows0[r, pl.ds(c * L, L)] = zeros16
        return 0

    lax.fori_loop(0, CHUNK, _zero_rows, 0)

    rbase = sid * ROWS_PER_TILE
    for k in range(ROWS_PER_TILE // CHUNK):
        pltpu.sync_copy(rows0, acc.at[pl.ds(rbase + k * CHUNK, CHUNK)])
    plsc.subcore_barrier()

    cbase = wid * CHUNKS_PER_W

    def _fire_idx(c, j):
        pltpu.async_copy(idx_hbm.at[cbase + c], idxb[j], isems[j])
        pltpu.async_copy(w_hbm.at[cbase + c], wvb[j], isems[j])

    def _wait_idx(c, j):
        pltpu.make_async_copy(idx_hbm.at[cbase + c], idxb[j], isems[j]).wait()
        pltpu.make_async_copy(w_hbm.at[cbase + c], wvb[j], isems[j]).wait()

    def _fire_gather(j):
        pltpu.async_copy(x_hbm.at[idxb[j].at[0]], rowsb[j], gsems[j])

    def _wait_gather(j):
        pltpu.make_async_copy(x_hbm.at[idxb[j].at[0]], rowsb[j],
                              gsems[j]).wait()

    def _scale(j):
        rows_v = rowsb[j]
        w_v = wvb[j]
        for g in range(CHUNK // L):
            w16 = w_v[pl.ds(g * L, L)]
            for e in range(L):
                wb = jnp.broadcast_to(w16[e], (L,))
                r = g * L + e
                for cc in range(D // L):
                    rows_v[r, pl.ds(cc * L, L)] = (
                        rows_v[r, pl.ds(cc * L, L)] * wb
                    )

    def _snap_dst(j):
        for g in range(CHUNK // L):
            dstb[j][0, pl.ds(g * L, L)] = idxb[j][1, pl.ds(g * L, L)]

    def _fire_scatter(j):
        pltpu.async_copy(rowsb[j], acc.at[dstb[j].at[0]], ssems[j], add=True)

    def _wait_scatter(j):
        pltpu.make_async_copy(rowsb[j], acc.at[dstb[j].at[0]],
                              ssems[j]).wait()

    def _chunk(c, j, wait_sc, fire_g, refill):
        jp = (j + 1) % NR
        if fire_g:
            _wait_idx(c + 1, jp)
            if wait_sc:
                _wait_scatter(jp)
            _fire_gather(jp)
        _wait_gather(j)
        _scale(j)
        _snap_dst(j)
        _fire_scatter(j)
        if refill:
            _fire_idx(jnp.minimum(c + 3, LAST), j)

    for j in range(NR):
        _fire_idx(j, j)
    _wait_idx(0, 0)
    _fire_gather(0)

    _chunk(0, 0, wait_sc=False, fire_g=True, refill=True)
    _chunk(1, 1, wait_sc=False, fire_g=True, refill=True)
    _chunk(2, 2, wait_sc=True, fire_g=True, refill=True)

    def _outer(o, _):
        c0 = 3 + o * 3
        for j in range(NR):
            _chunk(c0 + j, j, wait_sc=True, fire_g=True, refill=True)
        return 0

    lax.fori_loop(0, 40, _outer, 0)

    _chunk(123, 0, wait_sc=True, fire_g=True, refill=False)
    _chunk(124, 1, wait_sc=False, fire_g=False, refill=False)

    _wait_idx(LAST, 2)
    for j in range(NR):
        _wait_scatter(j)
    plsc.subcore_barrier()

    pltpu.sync_copy(acc.at[pl.ds(rbase, ROWS_PER_TILE)],
                    out_hbm.at[cid, pl.ds(rbase, ROWS_PER_TILE)])


_BLK = 1000


def _mm_body(p_ref, w_ref, b_ref, o_ref):
    s = p_ref[0] + p_ref[1]
    o_ref[...] = (
        jnp.dot(s, w_ref[...], preferred_element_type=jnp.float32) + b_ref[...]
    )


_matmul = pl.pallas_call(
    _mm_body,
    grid=(N // _BLK,),
    in_specs=[
        pl.BlockSpec((NC, _BLK, D), lambda i: (0, i, 0)),
        pl.BlockSpec((D, D), lambda i: (0, 0)),
        pl.BlockSpec((1, D), lambda i: (0, 0)),
    ],
    out_specs=pl.BlockSpec((_BLK, D), lambda i: (i, 0)),
    out_shape=jax.ShapeDtypeStruct((N, D), jnp.float32),
)


def kernel(x, edge_index, edge_weight, W, b):
    nwc = NW * CHUNKS_PER_W
    idx = edge_index.reshape(2, nwc, CHUNK).transpose(1, 0, 2)
    wc = edge_weight.reshape(nwc, CHUNK)
    partials = _aggregate(x, idx, wc)
    return _matmul(partials, W, b.reshape(1, D))


# speedup vs baseline: 10.0264x; 1.0556x over previous
import functools

import jax
import jax.numpy as jnp
from jax import lax
from jax.experimental import pallas as pl
from jax.experimental.pallas import tpu as pltpu
from jax.experimental.pallas import tpu_sc as plsc

N = 10000
D = 128
E = 320000
NC = 2
NS = 16
L = 16
NW = NC * NS
CHUNK = 80
CHUNKS_PER_W = 125
EDGES_PER_W = CHUNK * CHUNKS_PER_W
N_PAD = 10240
ROWS_PER_TILE = N_PAD // NS
NR = 4
LAST = CHUNKS_PER_W - 1

_mesh = plsc.VectorSubcoreMesh(core_axis_name="c", subcore_axis_name="s")


@functools.partial(
    pl.kernel,
    mesh=_mesh,
    out_type=jax.ShapeDtypeStruct((NC, N_PAD, D), jnp.float32),
    scratch_types=[
        pltpu.VMEM((2, CHUNK), jnp.int32),
        pltpu.VMEM((2, CHUNK), jnp.int32),
        pltpu.VMEM((2, CHUNK), jnp.int32),
        pltpu.VMEM((2, CHUNK), jnp.int32),
        pltpu.VMEM((CHUNK,), jnp.float32),
        pltpu.VMEM((CHUNK,), jnp.float32),
        pltpu.VMEM((CHUNK,), jnp.float32),
        pltpu.VMEM((CHUNK,), jnp.float32),
        pltpu.VMEM((1, CHUNK), jnp.int32),
        pltpu.VMEM((1, CHUNK), jnp.int32),
        pltpu.VMEM((1, CHUNK), jnp.int32),
        pltpu.VMEM((1, CHUNK), jnp.int32),
        pltpu.VMEM((CHUNK, D), jnp.float32),
        pltpu.VMEM((CHUNK, D), jnp.float32),
        pltpu.VMEM((CHUNK, D), jnp.float32),
        pltpu.VMEM((CHUNK, D), jnp.float32),
        pltpu.VMEM_SHARED((N_PAD, D), jnp.float32),
        pltpu.SemaphoreType.DMA,
        pltpu.SemaphoreType.DMA,
        pltpu.SemaphoreType.DMA,
        pltpu.SemaphoreType.DMA,
        pltpu.SemaphoreType.DMA,
        pltpu.SemaphoreType.DMA,
        pltpu.SemaphoreType.DMA,
        pltpu.SemaphoreType.DMA,
        pltpu.SemaphoreType.DMA,
        pltpu.SemaphoreType.DMA,
        pltpu.SemaphoreType.DMA,
        pltpu.SemaphoreType.DMA,
    ],
)
def _aggregate(x_hbm, idx_hbm, w_hbm, out_hbm,
               idx0, idx1, idx2, idx3, w0, w1, w2, w3,
               dst0, dst1, dst2, dst3, rows0, rows1, rows2, rows3, acc,
               gsem0, gsem1, gsem2, gsem3, isem0, isem1, isem2, isem3,
               ssem0, ssem1, ssem2, ssem3):
    cid = lax.axis_index("c")
    sid = lax.axis_index("s")
    wid = cid * NS + sid
    idxb = (idx0, idx1, idx2, idx3)
    wvb = (w0, w1, w2, w3)
    dstb = (dst0, dst1, dst2, dst3)
    rowsb = (rows0, rows1, rows2, rows3)
    isems = (isem0, isem1, isem2, isem3)
    gsems = (gsem0, gsem1, gsem2, gsem3)
    ssems = (ssem0, ssem1, ssem2, ssem3)

    zeros16 = jnp.zeros((L,), jnp.float32)

    def _zero_rows(r, _):
        for c in range(D // L):
            rows0[r, pl.ds(c * L, L)] = zeros16
        return 0

    lax.fori_loop(0, CHUNK, _zero_rows, 0)

    rbase = sid * ROWS_PER_TILE
    for k in range(ROWS_PER_TILE // CHUNK):
        pltpu.sync_copy(rows0, acc.at[pl.ds(rbase + k * CHUNK, CHUNK)])
    plsc.subcore_barrier()

    cbase = wid * CHUNKS_PER_W

    def _fire_idx(c, j):
        pltpu.async_copy(idx_hbm.at[cbase + c], idxb[j], isems[j])
        pltpu.async_copy(w_hbm.at[cbase + c], wvb[j], isems[j])

    def _wait_idx(c, j):
        pltpu.make_async_copy(idx_hbm.at[cbase + c], idxb[j], isems[j]).wait()
        pltpu.make_async_copy(w_hbm.at[cbase + c], wvb[j], isems[j]).wait()

    def _fire_gather(j):
        pltpu.async_copy(x_hbm.at[idxb[j].at[0]], rowsb[j], gsems[j])

    def _wait_gather(j):
        pltpu.make_async_copy(x_hbm.at[idxb[j].at[0]], rowsb[j],
                              gsems[j]).wait()

    def _scale(j):
        rows_v = rowsb[j]
        w_v = wvb[j]
        for g in range(CHUNK // L):
            w16 = w_v[pl.ds(g * L, L)]
            for e in range(L):
                wb = jnp.broadcast_to(w16[e], (L,))
                r = g * L + e
                for cc in range(D // L):
                    rows_v[r, pl.ds(cc * L, L)] = (
                        rows_v[r, pl.ds(cc * L, L)] * wb
                    )

    def _snap_dst(j):
        for g in range(CHUNK // L):
            dstb[j][0, pl.ds(g * L, L)] = idxb[j][1, pl.ds(g * L, L)]

    def _fire_scatter(j):
        pltpu.async_copy(rowsb[j], acc.at[dstb[j].at[0]], ssems[j], add=True)

    def _wait_scatter(j):
        pltpu.make_async_copy(rowsb[j], acc.at[dstb[j].at[0]],
                              ssems[j]).wait()

    def _chunk(c, j):
        j2 = (j + 2) % NR
        _wait_idx(jnp.minimum(c + 2, LAST), j2)
        _wait_scatter(j2)
        _fire_gather(j2)
        _wait_gather(j)
        _scale(j)
        _snap_dst(j)
        _fire_scatter(j)
        _fire_idx(jnp.minimum(c + NR, LAST), j)

    zidx16 = jnp.zeros((L,), jnp.int32)
    for j in (2, 3):
        def _zero_rows_j(r, _, _rv=rowsb[j]):
            for cc in range(D // L):
                _rv[r, pl.ds(cc * L, L)] = zeros16
            return 0

        lax.fori_loop(0, CHUNK, _zero_rows_j, 0)
        for g in range(CHUNK // L):
            dstb[j][0, pl.ds(g * L, L)] = zidx16
        _fire_scatter(j)

    for j in range(NR):
        _fire_idx(j, j)
    _wait_idx(0, 0)
    _fire_gather(0)
    _wait_idx(1, 1)
    _fire_gather(1)

    def _outer(o, _):
        c0 = o * 4
        for j in range(NR):
            _chunk(c0 + j, j)
        return 0

    lax.fori_loop(0, 31, _outer, 0)

    _wait_gather(0)
    _scale(0)
    _snap_dst(0)
    _fire_scatter(0)

    _wait_idx(LAST, 2)
    _wait_idx(LAST, 3)
    _wait_gather(1)
    _wait_scatter(0)
    _wait_scatter(2)
    _wait_scatter(3)
    plsc.subcore_barrier()

    pltpu.sync_copy(acc.at[pl.ds(rbase, ROWS_PER_TILE)],
                    out_hbm.at[cid, pl.ds(rbase, ROWS_PER_TILE)])


_BLK = 1000


def _mm_body(p_ref, w_ref, b_ref, o_ref):
    s = p_ref[0] + p_ref[1]
    o_ref[...] = (
        jnp.dot(s, w_ref[...], preferred_element_type=jnp.float32) + b_ref[...]
    )


_matmul = pl.pallas_call(
    _mm_body,
    grid=(N // _BLK,),
    in_specs=[
        pl.BlockSpec((NC, _BLK, D), lambda i: (0, i, 0)),
        pl.BlockSpec((D, D), lambda i: (0, 0)),
        pl.BlockSpec((1, D), lambda i: (0, 0)),
    ],
    out_specs=pl.BlockSpec((_BLK, D), lambda i: (i, 0)),
    out_shape=jax.ShapeDtypeStruct((N, D), jnp.float32),
)


def kernel(x, edge_index, edge_weight, W, b):
    nwc = NW * CHUNKS_PER_W
    idx = edge_index.reshape(2, nwc, CHUNK).transpose(1, 0, 2)
    wc = edge_weight.reshape(nwc, CHUNK)
    partials = _aggregate(x, idx, wc)
    return _matmul(partials, W, b.reshape(1, D))
